# baseline (speedup 1.0000x reference)
import functools
import jax
import jax.numpy as jnp
from jax.experimental import pallas as pl


_NUM_GRAPHS = 64
_RATIO = 0.5


def _linear_body(x_ref, w_ref, b_ref, o_ref, *, relu):
    y = jnp.dot(x_ref[...], w_ref[...], preferred_element_type=jnp.float32)
    y = y + b_ref[...]
    if relu:
        y = jnp.maximum(y, 0.0)
    o_ref[...] = y


def _linear(x, Wt, b, relu=False, mb=2000):
    M, K = x.shape
    N = Wt.shape[1]
    if M % mb != 0:
        mb = M
    grid = (M // mb,)
    return pl.pallas_call(
        functools.partial(_linear_body, relu=relu),
        grid=grid,
        in_specs=[
            pl.BlockSpec((mb, K), lambda i: (i, 0)),
            pl.BlockSpec((K, N), lambda i: (0, 0)),
            pl.BlockSpec((1, N), lambda i: (0, 0)),
        ],
        out_specs=pl.BlockSpec((mb, N), lambda i: (i, 0)),
        out_shape=jax.ShapeDtypeStruct((M, N), jnp.float32),
    )(x, Wt, b.reshape(1, N))


def _conv_body(a_ref, x_ref, wr_ref, wo_ref, b_ref, o_ref, *, relu):
    y = jnp.dot(a_ref[...], wr_ref[...], preferred_element_type=jnp.float32)
    y = y + b_ref[...]
    y = y + jnp.dot(x_ref[...], wo_ref[...], preferred_element_type=jnp.float32)
    if relu:
        y = jnp.maximum(y, 0.0)
    o_ref[...] = y


def _conv_mm(agg, x, Wrel_t, Wroot_t, b, relu=True, mb=2000):
    M, Ka = agg.shape
    Kx = x.shape[1]
    N = Wrel_t.shape[1]
    if M % mb != 0:
        mb = M
    grid = (M // mb,)
    return pl.pallas_call(
        functools.partial(_conv_body, relu=relu),
        grid=grid,
        in_specs=[
            pl.BlockSpec((mb, Ka), lambda i: (i, 0)),
            pl.BlockSpec((mb, Kx), lambda i: (i, 0)),
            pl.BlockSpec((Ka, N), lambda i: (0, 0)),
            pl.BlockSpec((Kx, N), lambda i: (0, 0)),
            pl.BlockSpec((1, N), lambda i: (0, 0)),
        ],
        out_specs=pl.BlockSpec((mb, N), lambda i: (i, 0)),
        out_shape=jax.ShapeDtypeStruct((M, N), jnp.float32),
    )(agg, x, Wrel_t, Wroot_t, b.reshape(1, N))


def _topk_mask(score, batch, alive, ratio, num_graphs):
    neg = jnp.finfo(score.dtype).min
    s = jnp.where(alive, score, neg)
    order = jnp.lexsort((-s, batch))
    counts = jnp.bincount(batch, length=num_graphs)
    offsets = jnp.cumsum(counts) - counts
    pos = jnp.arange(batch.shape[0]) - offsets[batch[order]]
    alive_cnt = jax.ops.segment_sum(alive.astype(jnp.int32), batch, num_segments=num_graphs)
    k = jnp.ceil(ratio * alive_cnt.astype(jnp.float32)).astype(jnp.int32)
    sel = pos < k[batch[order]]
    return jnp.zeros_like(alive).at[order].set(sel)


def _mean_pool(xs, batch, alive, num_graphs):
    s = jax.ops.segment_sum(xs, batch, num_segments=num_graphs)
    c = jax.ops.segment_sum(alive.astype(xs.dtype), batch, num_segments=num_graphs)
    return s / jnp.maximum(c, 1.0)[:, None]


def _score(g, agg, relW, relb, rootW):
    return (agg @ relW.T + relb + g @ rootW.T).reshape(-1)


def kernel(x, edge_index, edge_weight, batch, params):
    p = params
    n = x.shape[0]
    src, dst = edge_index[0], edge_index[1]
    alive0 = jnp.ones((n,), dtype=bool)
    e_mask0 = jnp.ones_like(edge_weight)

    h = _linear(x, p['W1'].T, p['b1'], relu=True)

    agg1 = jax.ops.segment_sum(h[src] * edge_weight[:, None], dst, num_segments=n)
    g1 = _conv_mm(agg1, h, p['g1_rel_W'].T, p['g1_root_W'].T, p['g1_rel_b'], relu=True)

    gath1 = g1[src]
    sagg1 = jax.ops.segment_sum(gath1 * e_mask0[:, None], dst, num_segments=n)
    s1 = _score(g1, sagg1, p['p1_rel_W'], p['p1_rel_b'], p['p1_root_W'])
    alive1 = _topk_mask(s1, batch, alive0, _RATIO, _NUM_GRAPHS)
    m1 = alive1.astype(x.dtype)
    t1 = jnp.tanh(s1)
    x1 = g1 * t1[:, None] * m1[:, None]
    e_mask1 = e_mask0 * m1[src] * m1[dst]
    ew1 = edge_weight * e_mask1
    gp1 = _mean_pool(x1, batch, alive1, _NUM_GRAPHS)

    msg2 = ((gath1 * t1[src][:, None]) * m1[src][:, None]) * ew1[:, None]
    agg2 = jax.ops.segment_sum(msg2, dst, num_segments=n)
    g2 = _conv_mm(agg2, x1, p['g2_rel_W'].T, p['g2_root_W'].T, p['g2_rel_b'], relu=True)

    gath2 = g2[src]
    sagg2 = jax.ops.segment_sum(gath2 * e_mask1[:, None], dst, num_segments=n)
    s2 = _score(g2, sagg2, p['p2_rel_W'], p['p2_rel_b'], p['p2_root_W'])
    alive2 = _topk_mask(s2, batch, alive1, _RATIO, _NUM_GRAPHS)
    m2 = alive2.astype(x.dtype)
    t2 = jnp.tanh(s2)
    x2 = g2 * t2[:, None] * m2[:, None]
    e_mask2 = e_mask1 * m2[src] * m2[dst]
    ew2 = edge_weight * e_mask2
    gp2 = _mean_pool(x2, batch, alive2, _NUM_GRAPHS)

    msg3 = ((gath2 * t2[src][:, None]) * m2[src][:, None]) * ew2[:, None]
    agg3 = jax.ops.segment_sum(msg3, dst, num_segments=n)
    g3 = _conv_mm(agg3, x2, p['g3_rel_W'].T, p['g3_root_W'].T, p['g3_rel_b'], relu=True)

    sagg3 = jax.ops.segment_sum(g3[src] * e_mask2[:, None], dst, num_segments=n)
    s3 = _score(g3, sagg3, p['p3_rel_W'], p['p3_rel_b'], p['p3_root_W'])
    alive3 = _topk_mask(s3, batch, alive2, _RATIO, _NUM_GRAPHS)
    m3 = alive3.astype(x.dtype)
    x3 = g3 * jnp.tanh(s3)[:, None] * m3[:, None]
    gp3 = _mean_pool(x3, batch, alive3, _NUM_GRAPHS)

    out = gp1 + gp2 + gp3
    out = _linear(out, p['m1_W'].T, p['m1_b'], relu=True)
    out = _linear(out, p['m2_W'].T, p['m2_b'], relu=False)
    return out


# speedup vs baseline: 1.0459x; 1.0459x over previous
import functools
import jax
import jax.numpy as jnp
from jax.experimental import pallas as pl


_NUM_GRAPHS = 64
_RATIO = 0.5


def _linear_body(x_ref, w_ref, b_ref, o_ref, *, relu):
    y = jnp.dot(x_ref[...], w_ref[...], preferred_element_type=jnp.float32)
    y = y + b_ref[...]
    if relu:
        y = jnp.maximum(y, 0.0)
    o_ref[...] = y


def _linear(x, Wt, b, relu=False, mb=2000):
    M, K = x.shape
    N = Wt.shape[1]
    if M % mb != 0:
        mb = M
    grid = (M // mb,)
    return pl.pallas_call(
        functools.partial(_linear_body, relu=relu),
        grid=grid,
        in_specs=[
            pl.BlockSpec((mb, K), lambda i: (i, 0)),
            pl.BlockSpec((K, N), lambda i: (0, 0)),
            pl.BlockSpec((1, N), lambda i: (0, 0)),
        ],
        out_specs=pl.BlockSpec((mb, N), lambda i: (i, 0)),
        out_shape=jax.ShapeDtypeStruct((M, N), jnp.float32),
    )(x, Wt, b.reshape(1, N))


def _conv_body(a_ref, x_ref, wr_ref, wo_ref, b_ref, o_ref, *, relu):
    y = jnp.dot(a_ref[...], wr_ref[...], preferred_element_type=jnp.float32)
    y = y + b_ref[...]
    y = y + jnp.dot(x_ref[...], wo_ref[...], preferred_element_type=jnp.float32)
    if relu:
        y = jnp.maximum(y, 0.0)
    o_ref[...] = y


def _conv_mm(agg, x, Wrel_t, Wroot_t, b, relu=True, mb=2000):
    M, Ka = agg.shape
    Kx = x.shape[1]
    N = Wrel_t.shape[1]
    if M % mb != 0:
        mb = M
    grid = (M // mb,)
    return pl.pallas_call(
        functools.partial(_conv_body, relu=relu),
        grid=grid,
        in_specs=[
            pl.BlockSpec((mb, Ka), lambda i: (i, 0)),
            pl.BlockSpec((mb, Kx), lambda i: (i, 0)),
            pl.BlockSpec((Ka, N), lambda i: (0, 0)),
            pl.BlockSpec((Kx, N), lambda i: (0, 0)),
            pl.BlockSpec((1, N), lambda i: (0, 0)),
        ],
        out_specs=pl.BlockSpec((mb, N), lambda i: (i, 0)),
        out_shape=jax.ShapeDtypeStruct((M, N), jnp.float32),
    )(agg, x, Wrel_t, Wroot_t, b.reshape(1, N))


def _topk_mask(score, batch, alive, ratio, num_graphs):
    neg = jnp.finfo(score.dtype).min
    s = jnp.where(alive, score, neg)
    order = jnp.lexsort((-s, batch))
    counts = jnp.bincount(batch, length=num_graphs)
    offsets = jnp.cumsum(counts) - counts
    pos = jnp.arange(batch.shape[0]) - offsets[batch[order]]
    alive_cnt = jax.ops.segment_sum(alive.astype(jnp.int32), batch, num_segments=num_graphs)
    k = jnp.ceil(ratio * alive_cnt.astype(jnp.float32)).astype(jnp.int32)
    sel = pos < k[batch[order]]
    return jnp.zeros_like(alive).at[order].set(sel)


def _mean_pool(xs, batch, alive, num_graphs):
    s = jax.ops.segment_sum(xs, batch, num_segments=num_graphs)
    c = jax.ops.segment_sum(alive.astype(xs.dtype), batch, num_segments=num_graphs)
    return s / jnp.maximum(c, 1.0)[:, None]


def _score(g, agg, relW, relb, rootW):
    return (agg @ relW.T + relb + g @ rootW.T).reshape(-1)


def kernel(x, edge_index, edge_weight, batch, params):
    p = params
    n = x.shape[0]
    src, dst = edge_index[0], edge_index[1]
    alive0 = jnp.ones((n,), dtype=bool)
    e_mask0 = jnp.ones_like(edge_weight)

    h = _linear(x, p['W1'].T, p['b1'], relu=True)

    agg1 = jax.ops.segment_sum(h[src] * edge_weight[:, None], dst, num_segments=n)
    g1 = _conv_mm(agg1, h, p['g1_rel_W'].T, p['g1_root_W'].T, p['g1_rel_b'], relu=True)

    sagg1 = jax.ops.segment_sum(g1[src] * e_mask0[:, None], dst, num_segments=n)
    s1 = _score(g1, sagg1, p['p1_rel_W'], p['p1_rel_b'], p['p1_root_W'])
    alive1 = _topk_mask(s1, batch, alive0, _RATIO, _NUM_GRAPHS)
    m1 = alive1.astype(x.dtype)
    x1 = g1 * jnp.tanh(s1)[:, None] * m1[:, None]
    e_mask1 = e_mask0 * m1[src] * m1[dst]
    ew1 = edge_weight * e_mask1
    gp1 = _mean_pool(x1, batch, alive1, _NUM_GRAPHS)

    agg2 = jax.ops.segment_sum(x1[src] * ew1[:, None], dst, num_segments=n)
    g2 = _conv_mm(agg2, x1, p['g2_rel_W'].T, p['g2_root_W'].T, p['g2_rel_b'], relu=True)

    sagg2 = jax.ops.segment_sum(g2[src] * e_mask1[:, None], dst, num_segments=n)
    s2 = _score(g2, sagg2, p['p2_rel_W'], p['p2_rel_b'], p['p2_root_W'])
    alive2 = _topk_mask(s2, batch, alive1, _RATIO, _NUM_GRAPHS)
    m2 = alive2.astype(x.dtype)
    x2 = g2 * jnp.tanh(s2)[:, None] * m2[:, None]
    e_mask2 = e_mask1 * m2[src] * m2[dst]
    ew2 = edge_weight * e_mask2
    gp2 = _mean_pool(x2, batch, alive2, _NUM_GRAPHS)

    agg3 = jax.ops.segment_sum(x2[src] * ew2[:, None], dst, num_segments=n)
    g3 = _conv_mm(agg3, x2, p['g3_rel_W'].T, p['g3_root_W'].T, p['g3_rel_b'], relu=True)

    sagg3 = jax.ops.segment_sum(g3[src] * e_mask2[:, None], dst, num_segments=n)
    s3 = _score(g3, sagg3, p['p3_rel_W'], p['p3_rel_b'], p['p3_root_W'])
    alive3 = _topk_mask(s3, batch, alive2, _RATIO, _NUM_GRAPHS)
    m3 = alive3.astype(x.dtype)
    x3 = g3 * jnp.tanh(s3)[:, None] * m3[:, None]
    gp3 = _mean_pool(x3, batch, alive3, _NUM_GRAPHS)

    out = gp1 + gp2 + gp3
    out = _linear(out, p['m1_W'].T, p['m1_b'], relu=True)
    out = _linear(out, p['m2_W'].T, p['m2_b'], relu=False)
    return out
